# baseline (speedup 1.0000x reference)
import functools

import jax
import jax.numpy as jnp
from jax import lax
from jax.experimental import pallas as pl
from jax.experimental.pallas import tpu as pltpu
from jax.experimental.pallas import tpu_sc as plsc

_B = 16384
_D = 64
_TR = 8
_NTILES = 1000000 // _TR
_NC = 2
_NS = 16
_NW = _NC * _NS
_BPW = _B // _NW
_PASS = 128
_NPASS = _BPW // _PASS


def _distmult_body(hidx_hbm, tidx_hbm, ridx_hbm, entity_hbm, rel_hbm,
                   out_hbm, hidx, tidx, ridx,
                   h_rows, t_rows, r_rows, out_buf, sem):
    wid = lax.axis_index("s") * _NC + lax.axis_index("c")

    pltpu.sync_copy(hidx_hbm.at[wid], hidx)
    pltpu.sync_copy(tidx_hbm.at[wid], tidx)
    pltpu.sync_copy(ridx_hbm.at[wid], ridx)

    iota16 = lax.iota(jnp.int32, 16)

    def issue(p, par):
        row0 = p * _PASS
        for q in range(_PASS // 16):
            sl = pl.ds(row0 + q * 16, 16)
            hv, tv, rv = hidx[sl], tidx[sl], ridx[sl]
            for i in range(16):
                lj = q * 16 + i
                a, b = lj >> 3, lj & 7
                hh, tt, rr = hv[i], tv[i], rv[i]
                pltpu.async_copy(entity_hbm.at[hh >> 3, hh & 7],
                                 h_rows.at[par, a, b], sem)
                pltpu.async_copy(entity_hbm.at[tt >> 3, tt & 7],
                                 t_rows.at[par, a, b], sem)
                pltpu.async_copy(rel_hbm.at[rr >> 3, rr & 7],
                                 r_rows.at[par, a, b], sem)

    issue(0, 0)

    def do_pass(p, carry):
        par = p & 1

        @pl.when(p < _NPASS - 1)
        def _():
            issue(p + 1, 1 - par)

        src = entity_hbm.at[pl.ds(0, 16)]
        rsrc = rel_hbm.at[pl.ds(0, 16)]
        pltpu.make_async_copy(src, h_rows.at[par], sem).wait()
        pltpu.make_async_copy(src, t_rows.at[par], sem).wait()
        pltpu.make_async_copy(rsrc, r_rows.at[par], sem).wait()

        row0 = p * _PASS
        for g in range(_PASS // 16):
            tot = jnp.zeros((16,), jnp.float32)
            for jj in range(16):
                lj = g * 16 + jj
                a, b = lj >> 3, lj & 7
                acc = None
                for c in range(_D // 16):
                    sl = pl.ds(c * 16, 16)
                    prod = (h_rows[par, a, b, sl] * r_rows[par, a, b, sl]
                            * t_rows[par, a, b, sl])
                    acc = prod if acc is None else acc + prod
                tot = jnp.where(iota16 == jj, jnp.sum(acc), tot)
            out_buf[pl.ds(row0 + g * 16, 16)] = tot
        return carry

    lax.fori_loop(0, _NPASS, do_pass, 0)

    pltpu.sync_copy(out_buf, out_hbm.at[pl.ds(wid * _BPW, _BPW)])


_distmult_sc = functools.partial(
    pl.kernel,
    out_type=jax.ShapeDtypeStruct((_B,), jnp.float32),
    scratch_types=[
        pltpu.VMEM((_BPW,), jnp.int32),
        pltpu.VMEM((_BPW,), jnp.int32),
        pltpu.VMEM((_BPW,), jnp.int32),
        pltpu.VMEM((2, _PASS // _TR, _TR, _D), jnp.float32),
        pltpu.VMEM((2, _PASS // _TR, _TR, _D), jnp.float32),
        pltpu.VMEM((2, _PASS // _TR, _TR, _D), jnp.float32),
        pltpu.VMEM((_BPW,), jnp.float32),
        pltpu.SemaphoreType.DMA,
    ],
    mesh=plsc.VectorSubcoreMesh(core_axis_name="c", subcore_axis_name="s"),
    compiler_params=pltpu.CompilerParams(needs_layout_passes=False),
)(_distmult_body)


@jax.jit
def kernel(head_idx, rel_idx, tail_idx, entity_table, relation_table):
    h2, r2, t2 = (x.reshape(_NW, _BPW) for x in (head_idx, rel_idx, tail_idx))
    et3 = entity_table.reshape(_NTILES, _TR, _D)
    rt3 = relation_table.reshape(1000 // _TR, _TR, _D)
    return _distmult_sc(h2, t2, r2, et3, rt3)
